# Initial kernel scaffold; baseline (speedup 1.0000x reference)
import jax
import jax.numpy as jnp
from jax.experimental import pallas as pl


def kernel(x, edge_index, W_l1, b_l1, W_r1, gamma, beta, W_l2, b_l2, W_r2):
    raise NotImplementedError("write your pallas kernel here")


# speedup vs baseline: 2.9740x; 2.9740x over previous
import jax
import jax.numpy as jnp
from jax import lax
from jax.experimental import pallas as pl
from jax.experimental.pallas import tpu as pltpu
from jax.experimental.pallas import tpu_sc as plsc

N = 10000
D = 128
E = 320000
EPS = 1e-5

NC = 2
NS = 16
NW = NC * NS

NP = 10112
EP = 327680
C = 128
EDGES_PER_TILE = EP // NW
CHUNKS = EDGES_PER_TILE // C
ROWS_PER_TILE = NP // NS
WB_CHUNKS = ((0, 128), (128, 128), (256, 128), (384, 128), (512, 120))


def _sc_body(do_deg, x_hbm, src_hbm, dst_hbm, zrows_hbm, ones_hbm,
             out_hbm, deg_out_hbm, src_idx, dst_idx, rows, ones_v,
             acc_sh, sem):
    c = lax.axis_index("c")
    s = lax.axis_index("s")
    wid = s * NC + c
    r0 = s * ROWS_PER_TILE
    ebase = wid * EDGES_PER_TILE

    def zero_acc():
        pltpu.sync_copy(zrows_hbm, rows)
        for off, nrows in WB_CHUNKS:
            pltpu.sync_copy(rows.at[pl.ds(0, nrows)],
                            acc_sh.at[pl.ds(r0 + off, nrows)])

    def writeback(dst_ref):
        for off, nrows in WB_CHUNKS:
            pltpu.sync_copy(acc_sh.at[pl.ds(r0 + off, nrows)],
                            rows.at[pl.ds(0, nrows)])
            pltpu.sync_copy(rows.at[pl.ds(0, nrows)],
                            dst_ref.at[c, pl.ds(r0 + off, nrows)])

    zero_acc()
    plsc.subcore_barrier()

    def chunk(i, carry):
        base = ebase + i * C
        pltpu.sync_copy(src_hbm.at[pl.ds(base, C)], src_idx)
        pltpu.sync_copy(dst_hbm.at[pl.ds(base, C)], dst_idx)
        pltpu.async_copy(x_hbm.at[src_idx], rows, sem).wait()
        pltpu.sync_copy(rows, acc_sh.at[dst_idx], add=True)
        return carry

    lax.fori_loop(0, CHUNKS, chunk, 0)
    plsc.subcore_barrier()
    writeback(out_hbm)

    if do_deg:
        plsc.subcore_barrier()
        zero_acc()
        pltpu.sync_copy(ones_hbm, ones_v)
        plsc.subcore_barrier()

        def dchunk(i, carry):
            base = ebase + i * C
            pltpu.sync_copy(dst_hbm.at[pl.ds(base, C)], dst_idx)
            pltpu.sync_copy(ones_v, acc_sh.at[dst_idx], add=True)
            return carry

        lax.fori_loop(0, CHUNKS, dchunk, 0)
        plsc.subcore_barrier()
        writeback(deg_out_hbm)


def _make_sc_agg(do_deg):
    mesh = plsc.VectorSubcoreMesh(core_axis_name="c", subcore_axis_name="s",
                                  num_cores=NC, num_subcores=NS)
    out_type = [jax.ShapeDtypeStruct((NC, NP, D), jnp.float32)]
    if do_deg:
        out_type.append(jax.ShapeDtypeStruct((NC, NP, D), jnp.float32))
    scratch = [
        pltpu.VMEM((C,), jnp.int32),
        pltpu.VMEM((C,), jnp.int32),
        pltpu.VMEM((C, D), jnp.float32),
        pltpu.VMEM((C, D), jnp.float32),
        pltpu.VMEM_SHARED((NP, D), jnp.float32),
        pltpu.SemaphoreType.DMA,
    ]
    if do_deg:
        def body(x, src, dst, zr, on, out, deg_out, *rest):
            return _sc_body(True, x, src, dst, zr, on, out, deg_out, *rest)
    else:
        def body(x, src, dst, zr, on, out, *rest):
            return _sc_body(False, x, src, dst, zr, on, out, None, *rest)
    return pl.kernel(body, out_type=tuple(out_type), mesh=mesh,
                     scratch_types=scratch)


def _tc_layer1(x_ref, parts_ref, degp_ref, wl_ref, bl_ref, wr_ref,
               gamma_ref, beta_ref, h_ref):
    agg = parts_ref[0, :N, :] + parts_ref[1, :N, :]
    deg16 = degp_ref[0, :N, :16] + degp_ref[1, :N, :16]
    deg = jnp.max(deg16, axis=1, keepdims=True)
    rdeg = 1.0 / jnp.maximum(deg, 1.0)
    dn = (((1,), (1,)), ((), ()))
    h = lax.dot_general(agg * rdeg, wl_ref[...], dn,
                        preferred_element_type=jnp.float32)
    h = h + bl_ref[...] + lax.dot_general(x_ref[...], wr_ref[...], dn,
                                          preferred_element_type=jnp.float32)
    mean = jnp.mean(h, axis=0, keepdims=True)
    var = jnp.mean((h - mean) ** 2, axis=0, keepdims=True)
    h_hat = (h - mean) * lax.rsqrt(var + EPS)
    h = gamma_ref[...] * h_hat + beta_ref[...]
    h_ref[...] = jnp.maximum(h, 0.0)


def _tc_layer2(h_ref, parts_ref, degp_ref, wl_ref, bl_ref, wr_ref, out_ref):
    agg = parts_ref[0, :N, :] + parts_ref[1, :N, :]
    deg16 = degp_ref[0, :N, :16] + degp_ref[1, :N, :16]
    deg = jnp.max(deg16, axis=1, keepdims=True)
    rdeg = 1.0 / jnp.maximum(deg, 1.0)
    dn = (((1,), (1,)), ((), ()))
    out = lax.dot_general(agg * rdeg, wl_ref[...], dn,
                          preferred_element_type=jnp.float32)
    out_ref[...] = out + bl_ref[...] + lax.dot_general(
        h_ref[...], wr_ref[...], dn, preferred_element_type=jnp.float32)


_sc_agg_deg = _make_sc_agg(True)
_sc_agg = _make_sc_agg(False)


def kernel(x, edge_index, W_l1, b_l1, W_r1, gamma, beta, W_l2, b_l2, W_r2):
    src = edge_index[0]
    dst = edge_index[1]
    npad = EP - E
    src_p = jnp.concatenate([src, jnp.zeros((npad,), jnp.int32)])
    dst_p = jnp.concatenate(
        [dst, N + (jnp.arange(npad, dtype=jnp.int32) % (NP - N))])
    zrows = jnp.zeros((C, D), jnp.float32)
    ones128 = jnp.ones((C, D), jnp.float32)

    parts1, degp = _sc_agg_deg(x, src_p, dst_p, zrows, ones128)

    bl1 = b_l1.reshape(1, D)
    g = gamma.reshape(1, D)
    b = beta.reshape(1, D)
    h = pl.pallas_call(
        _tc_layer1,
        out_shape=jax.ShapeDtypeStruct((N, D), jnp.float32),
    )(x, parts1, degp, W_l1, bl1, W_r1, g, b)

    (parts2,) = _sc_agg(h, src_p, dst_p, zrows, ones128)

    bl2 = b_l2.reshape(1, D)
    out = pl.pallas_call(
        _tc_layer2,
        out_shape=jax.ShapeDtypeStruct((N, D), jnp.float32),
    )(h, parts2, degp, W_l2, bl2, W_r2)
    return out
